# baseline (speedup 1.0000x reference)
import functools

import jax
import jax.numpy as jnp
from jax import lax
from jax.experimental import pallas as pl
from jax.experimental.pallas import tpu as pltpu
from jax.experimental.pallas import tpu_sc as plsc

N = 10000
E = 320000
F = 128
H = F // 2

NC = 2
NS = 16
NW = NC * NS

C = 80
CA = 128
CHA = 160
EPT = E // NS
EPT_PAD = CHA * CA
EPW = E // NW
CHD = EPW // C
RPT = 624
TAIL = N - NS * RPT

_mesh = plsc.VectorSubcoreMesh(
    core_axis_name="c", subcore_axis_name="s", num_cores=NC, num_subcores=NS
)


@functools.partial(
    pl.kernel,
    out_type=[
        jax.ShapeDtypeStruct((N,), jnp.float32),
        jax.ShapeDtypeStruct((N,), jnp.float32),
    ],
    mesh=_mesh,
    scratch_types=[
        pltpu.VMEM((CHD, C), jnp.int32),
        pltpu.VMEM((CHD, C), jnp.float32),
        pltpu.VMEM_SHARED((N,), jnp.float32),
    ],
)
def _deg_kernel(col_hbm, w_hbm, z_hbm, deg0, deg1, col_v, w_v, dacc):
    c = lax.axis_index("c")
    s = lax.axis_index("s")
    wid = c * NS + s

    @pl.when(s == 0)
    def _():
        pltpu.sync_copy(z_hbm, dacc)

    plsc.subcore_barrier()

    pltpu.sync_copy(col_hbm.at[wid], col_v)
    pltpu.sync_copy(w_hbm.at[wid], w_v)

    def chunk(j, carry):
        pltpu.sync_copy(w_v.at[j], dacc.at[col_v.at[j]], add=True)
        return carry

    lax.fori_loop(0, CHD, chunk, 0)
    plsc.subcore_barrier()

    @pl.when(s == 0)
    def _():
        @pl.when(c == 0)
        def _():
            pltpu.sync_copy(dacc, deg0)

        @pl.when(c == 1)
        def _():
            pltpu.sync_copy(dacc, deg1)


MCH = CHA // 4


def _agg_half(s, slab_hbm, slab_v, gbufs, sbuf, acc, gsem, lsem,
              g_hbm, z_hbm, p_hbm):
    pltpu.sync_copy(z_hbm.at[pl.ds(s * RPT, RPT)], acc.at[pl.ds(s * RPT, RPT)])

    @pl.when(s == NS - 1)
    def _():
        pltpu.sync_copy(z_hbm.at[pl.ds(NS * RPT, TAIL)],
                        acc.at[pl.ds(NS * RPT, TAIL)])

    plsc.subcore_barrier()

    def scale(src, dst, w_ref):
        @plsc.parallel_loop(0, CA // 16, unroll=2)
        def _(gi):
            w16 = plsc.bitcast(w_ref[pl.ds(gi * 16, 16)], jnp.float32)
            for k in range(16):
                sc = w16[k]
                base = gi * 16 + k
                for v in range(H // 16):
                    fsl = pl.ds(v * 16, 16)
                    dst[base, fsl] = src[base, fsl] * sc

    pltpu.async_copy(slab_hbm.at[s, 0], slab_v.at[0], lsem)
    pltpu.make_async_copy(slab_hbm.at[s, 0], slab_v.at[0], lsem).wait()
    for b in range(3):
        pltpu.async_copy(g_hbm.at[slab_v.at[0, b, 0]], gbufs[b], gsem)

    def ring(m, carry):
        p = m % 2
        q = (p + 1) % 2

        @pl.when(m + 1 < MCH)
        def _():
            pltpu.async_copy(slab_hbm.at[s, m + 1], slab_v.at[q], lsem)

        for b in range(4):
            j = 4 * m + b
            pltpu.make_async_copy(
                g_hbm.at[slab_v.at[p, b, 0]], gbufs[b], gsem).wait()

            if b == 1:
                @pl.when(m + 1 < MCH)
                def _():
                    pltpu.make_async_copy(
                        slab_hbm.at[s, m + 1], slab_v.at[q], lsem).wait()

            scale(gbufs[b], sbuf, slab_v.at[p, b, 2])
            pltpu.sync_copy(sbuf, acc.at[slab_v.at[p, b, 1]], add=True)

            nb = (b + 3) % 4
            if b == 0:
                pltpu.async_copy(
                    g_hbm.at[slab_v.at[p, 3, 0]], gbufs[nb], gsem)
            else:
                @pl.when(m + 1 < MCH)
                def _():
                    pltpu.async_copy(
                        g_hbm.at[slab_v.at[q, b - 1, 0]], gbufs[nb], gsem)
        return carry

    lax.fori_loop(0, MCH, ring, 0)
    plsc.subcore_barrier()

    sl = pl.ds(s * RPT, RPT)
    pltpu.sync_copy(acc.at[sl], p_hbm.at[sl])

    @pl.when(s == NS - 1)
    def _():
        tl = pl.ds(NS * RPT, TAIL)
        pltpu.sync_copy(acc.at[tl], p_hbm.at[tl])


@functools.partial(
    pl.kernel,
    out_type=[
        jax.ShapeDtypeStruct((N, H), jnp.float32),
        jax.ShapeDtypeStruct((N, H), jnp.float32),
    ],
    mesh=_mesh,
    scratch_types=[
        pltpu.VMEM((2, 4, 3, CA), jnp.int32),
        pltpu.VMEM((CA, H), jnp.float32),
        pltpu.VMEM((CA, H), jnp.float32),
        pltpu.VMEM((CA, H), jnp.float32),
        pltpu.VMEM((CA, H), jnp.float32),
        pltpu.VMEM((CA, H), jnp.float32),
        pltpu.VMEM_SHARED((N, H), jnp.float32),
        pltpu.SemaphoreType.DMA,
        pltpu.SemaphoreType.DMA,
    ],
    compiler_params=pltpu.CompilerParams(
        use_tc_tiling_on_sc=False, needs_layout_passes=False),
)
def _agg_kernel(slab_hbm, gl_hbm, gr_hbm, z_hbm, pl_out, pr_out,
                slab_v, g0, g1, g2, g3, sbuf, acc, gsem, lsem):
    c = lax.axis_index("c")
    s = lax.axis_index("s")

    gbufs = [g0, g1, g2, g3]

    @pl.when(c == 0)
    def _():
        _agg_half(s, slab_hbm, slab_v, gbufs, sbuf, acc, gsem, lsem,
                  gl_hbm, z_hbm, pl_out)

    @pl.when(c == 1)
    def _():
        _agg_half(s, slab_hbm, slab_v, gbufs, sbuf, acc, gsem, lsem,
                  gr_hbm, z_hbm, pr_out)


B = 10000


def _k2_body(d0_ref, d1_ref, x_ref, w1_ref, dis_ref, gl_ref, gr_ref):
    deg = d0_ref[...] + d1_ref[...] + 1.0
    dis = jnp.where(deg > 0, lax.rsqrt(deg), 0.0)
    dis_ref[...] = dis
    g0 = jnp.dot(x_ref[...], w1_ref[...],
                 preferred_element_type=jnp.float32) * dis
    gl_ref[...] = g0[:, :H]
    gr_ref[...] = g0[:, H:]


def _k4_body(al_ref, ar_ref, gl_ref, gr_ref, dis_ref, b1_ref, w2_ref,
             g1l_ref, g1r_ref):
    dis = dis_ref[...]
    hl = al_ref[...] + gl_ref[...]
    hr = ar_ref[...] + gr_ref[...]
    h1 = dis * jnp.concatenate([hl, hr], axis=1) + b1_ref[...]
    h1 = jnp.maximum(h1, 0.0)
    g1 = dis * jnp.dot(h1, w2_ref[...], preferred_element_type=jnp.float32)
    g1l_ref[...] = g1[:, :H]
    g1r_ref[...] = g1[:, H:]


def _k6_body(cl_ref, cr_ref, g1l_ref, g1r_ref, dis_ref, b2_ref,
             wfc1_ref, bfc1_ref, wfc2_ref, bfc2_ref, pred_ref, pc_ref):
    hl = cl_ref[...] + g1l_ref[...]
    hr = cr_ref[...] + g1r_ref[...]
    h2 = dis_ref[...] * jnp.concatenate([hl, hr], axis=1) + b2_ref[...]
    pred_ref[...] = (
        jnp.dot(h2, wfc1_ref[...], preferred_element_type=jnp.float32)
        + bfc1_ref[...]
    )
    pc_ref[...] = (
        jnp.dot(h2, wfc2_ref[...], preferred_element_type=jnp.float32)
        + bfc2_ref[...]
    )


def _row_block(minor):
    return pl.BlockSpec((B, minor), lambda i: (i, 0))


def _full_block(shape):
    return pl.BlockSpec(shape, lambda i: tuple(0 for _ in shape))


def kernel(x, edge_index, edge_weight, W1, b1, W2, b2, Wfc1, bfc1, Wfc2, bfc2):
    pad = EPT_PAD - EPT
    spread = jnp.broadcast_to(
        (jnp.arange(pad, dtype=jnp.int32) * 41) % N, (NS, pad))
    row_p = jnp.concatenate(
        [edge_index[0].reshape(NS, EPT), spread], axis=1)
    col_p = jnp.concatenate(
        [edge_index[1].reshape(NS, EPT), spread], axis=1)
    w_p = jnp.concatenate(
        [lax.bitcast_convert_type(edge_weight, jnp.int32).reshape(NS, EPT),
         jnp.zeros((NS, pad), jnp.int32)], axis=1)
    slab = jnp.stack([row_p, col_p, w_p], axis=1)
    slab = slab.reshape(NS, 3, MCH, 4, CA).transpose(0, 2, 3, 1, 4)
    col_w = edge_index[1].reshape(NW, CHD, C)
    w_w = edge_weight.reshape(NW, CHD, C)
    z_n = jnp.zeros((N,), jnp.float32)
    z_nh = jnp.zeros((N, H), jnp.float32)

    deg0, deg1 = _deg_kernel(col_w, w_w, z_n)

    grid = (N // B,)
    dis, g0l, g0r = pl.pallas_call(
        _k2_body,
        grid=grid,
        in_specs=[
            _row_block(1),
            _row_block(1),
            _row_block(F),
            _full_block((F, F)),
        ],
        out_specs=[_row_block(1), _row_block(H), _row_block(H)],
        out_shape=[
            jax.ShapeDtypeStruct((N, 1), jnp.float32),
            jax.ShapeDtypeStruct((N, H), jnp.float32),
            jax.ShapeDtypeStruct((N, H), jnp.float32),
        ],
    )(deg0[:, None], deg1[:, None], x, W1)

    a0l, a0r = _agg_kernel(slab, g0l, g0r, z_nh)

    g1l, g1r = pl.pallas_call(
        _k4_body,
        grid=grid,
        in_specs=[
            _row_block(H),
            _row_block(H),
            _row_block(H),
            _row_block(H),
            _row_block(1),
            _full_block((1, F)),
            _full_block((F, F)),
        ],
        out_specs=[_row_block(H), _row_block(H)],
        out_shape=[
            jax.ShapeDtypeStruct((N, H), jnp.float32),
            jax.ShapeDtypeStruct((N, H), jnp.float32),
        ],
    )(a0l, a0r, g0l, g0r, dis, b1[None, :], W2)

    a1l, a1r = _agg_kernel(slab, g1l, g1r, z_nh)

    pred, pred_cluster = pl.pallas_call(
        _k6_body,
        grid=grid,
        in_specs=[
            _row_block(H),
            _row_block(H),
            _row_block(H),
            _row_block(H),
            _row_block(1),
            _full_block((1, F)),
            _full_block((F, 64)),
            _full_block((1, 64)),
            _full_block((F, 16)),
            _full_block((1, 16)),
        ],
        out_specs=[_row_block(64), _row_block(16)],
        out_shape=[
            jax.ShapeDtypeStruct((N, 64), jnp.float32),
            jax.ShapeDtypeStruct((N, 16), jnp.float32),
        ],
    )(a1l, a1r, g1l, g1r, dis, b2[None, :], Wfc1, bfc1[None, :],
      Wfc2, bfc2[None, :])

    return (pred, pred_cluster)


# speedup vs baseline: 1.0725x; 1.0725x over previous
import functools

import jax
import jax.numpy as jnp
from jax import lax
from jax.experimental import pallas as pl
from jax.experimental.pallas import tpu as pltpu
from jax.experimental.pallas import tpu_sc as plsc

N = 10000
E = 320000
F = 128
H = F // 2

NC = 2
NS = 16
NW = NC * NS

C = 80
CA = 128
CHA = 160
EPT = E // NS
EPT_PAD = CHA * CA
EPW = E // NW
CHD = EPW // C
RPT = 624
TAIL = N - NS * RPT

_mesh = plsc.VectorSubcoreMesh(
    core_axis_name="c", subcore_axis_name="s", num_cores=NC, num_subcores=NS
)


@functools.partial(
    pl.kernel,
    out_type=[
        jax.ShapeDtypeStruct((N,), jnp.float32),
        jax.ShapeDtypeStruct((N,), jnp.float32),
    ],
    mesh=_mesh,
    scratch_types=[
        pltpu.VMEM((CHD, C), jnp.int32),
        pltpu.VMEM((CHD, C), jnp.float32),
        pltpu.VMEM_SHARED((N,), jnp.float32),
    ],
)
def _deg_kernel(col_hbm, w_hbm, z_hbm, deg0, deg1, col_v, w_v, dacc):
    c = lax.axis_index("c")
    s = lax.axis_index("s")
    wid = c * NS + s

    @pl.when(s == 0)
    def _():
        pltpu.sync_copy(z_hbm, dacc)

    plsc.subcore_barrier()

    pltpu.sync_copy(col_hbm.at[wid], col_v)
    pltpu.sync_copy(w_hbm.at[wid], w_v)

    def chunk(j, carry):
        pltpu.sync_copy(w_v.at[j], dacc.at[col_v.at[j]], add=True)
        return carry

    lax.fori_loop(0, CHD, chunk, 0)
    plsc.subcore_barrier()

    @pl.when(s == 0)
    def _():
        @pl.when(c == 0)
        def _():
            pltpu.sync_copy(dacc, deg0)

        @pl.when(c == 1)
        def _():
            pltpu.sync_copy(dacc, deg1)


MCH = CHA // 4


def _agg_half(s, slab_hbm, slab_v, gbufs, sbufs, acc, gsem, lsem, ssem,
              g_hbm, z_hbm, p_hbm):
    pltpu.sync_copy(z_hbm.at[pl.ds(s * RPT, RPT)], acc.at[pl.ds(s * RPT, RPT)])

    @pl.when(s == NS - 1)
    def _():
        pltpu.sync_copy(z_hbm.at[pl.ds(NS * RPT, TAIL)],
                        acc.at[pl.ds(NS * RPT, TAIL)])

    plsc.subcore_barrier()

    def scale(src, dst, w_ref):
        @plsc.parallel_loop(0, CA // 16, unroll=2)
        def _(gi):
            w16 = plsc.bitcast(w_ref[pl.ds(gi * 16, 16)], jnp.float32)
            for k in range(16):
                sc = w16[k]
                base = gi * 16 + k
                for v in range(H // 16):
                    fsl = pl.ds(v * 16, 16)
                    dst[base, fsl] = src[base, fsl] * sc

    pltpu.async_copy(slab_hbm.at[s, 0], slab_v.at[0], lsem)
    pltpu.make_async_copy(slab_hbm.at[s, 0], slab_v.at[0], lsem).wait()
    for b in range(3):
        pltpu.async_copy(g_hbm.at[slab_v.at[0, b, 0]], gbufs[b], gsem)

    def ring(m, carry):
        p = m % 2
        q = (p + 1) % 2

        @pl.when(m + 1 < MCH)
        def _():
            pltpu.async_copy(slab_hbm.at[s, m + 1], slab_v.at[q], lsem)

        for b in range(4):
            j = 4 * m + b
            pltpu.make_async_copy(
                g_hbm.at[slab_v.at[p, b, 0]], gbufs[b], gsem).wait()

            if b == 1:
                @pl.when(m + 1 < MCH)
                def _():
                    pltpu.make_async_copy(
                        slab_hbm.at[s, m + 1], slab_v.at[q], lsem).wait()

            @pl.when(j >= 2)
            def _():
                pltpu.make_async_copy(
                    sbufs[b % 2], acc.at[slab_v.at[p, b, 1]], ssem).wait()

            scale(gbufs[b], sbufs[b % 2], slab_v.at[p, b, 2])
            pltpu.async_copy(sbufs[b % 2], acc.at[slab_v.at[p, b, 1]],
                             ssem, add=True)

            nb = (b + 3) % 4
            if b == 0:
                pltpu.async_copy(
                    g_hbm.at[slab_v.at[p, 3, 0]], gbufs[nb], gsem)
            else:
                @pl.when(m + 1 < MCH)
                def _():
                    pltpu.async_copy(
                        g_hbm.at[slab_v.at[q, b - 1, 0]], gbufs[nb], gsem)
        return carry

    lax.fori_loop(0, MCH, ring, 0)

    pltpu.make_async_copy(sbufs[0], acc.at[slab_v.at[0, 0, 1]], ssem).wait()
    pltpu.make_async_copy(sbufs[1], acc.at[slab_v.at[0, 1, 1]], ssem).wait()
    plsc.subcore_barrier()

    sl = pl.ds(s * RPT, RPT)
    pltpu.sync_copy(acc.at[sl], p_hbm.at[sl])

    @pl.when(s == NS - 1)
    def _():
        tl = pl.ds(NS * RPT, TAIL)
        pltpu.sync_copy(acc.at[tl], p_hbm.at[tl])


@functools.partial(
    pl.kernel,
    out_type=[
        jax.ShapeDtypeStruct((N, H), jnp.float32),
        jax.ShapeDtypeStruct((N, H), jnp.float32),
    ],
    mesh=_mesh,
    scratch_types=[
        pltpu.VMEM((2, 4, 3, CA), jnp.int32),
        pltpu.VMEM((CA, H), jnp.float32),
        pltpu.VMEM((CA, H), jnp.float32),
        pltpu.VMEM((CA, H), jnp.float32),
        pltpu.VMEM((CA, H), jnp.float32),
        pltpu.VMEM((CA, H), jnp.float32),
        pltpu.VMEM((CA, H), jnp.float32),
        pltpu.VMEM_SHARED((N, H), jnp.float32),
        pltpu.SemaphoreType.DMA,
        pltpu.SemaphoreType.DMA,
        pltpu.SemaphoreType.DMA,
    ],
    compiler_params=pltpu.CompilerParams(
        use_tc_tiling_on_sc=False, needs_layout_passes=False),
)
def _agg_kernel(slab_hbm, gl_hbm, gr_hbm, z_hbm, pl_out, pr_out,
                slab_v, g0, g1, g2, g3, s0, s1, acc, gsem, lsem, ssem):
    c = lax.axis_index("c")
    s = lax.axis_index("s")

    gbufs = [g0, g1, g2, g3]
    sbufs = [s0, s1]

    @pl.when(c == 0)
    def _():
        _agg_half(s, slab_hbm, slab_v, gbufs, sbufs, acc, gsem, lsem, ssem,
                  gl_hbm, z_hbm, pl_out)

    @pl.when(c == 1)
    def _():
        _agg_half(s, slab_hbm, slab_v, gbufs, sbufs, acc, gsem, lsem, ssem,
                  gr_hbm, z_hbm, pr_out)


B = 2000


def _k2_body(d0_ref, d1_ref, x_ref, w1_ref, dis_ref, gl_ref, gr_ref):
    deg = d0_ref[...] + d1_ref[...] + 1.0
    dis = jnp.where(deg > 0, lax.rsqrt(deg), 0.0)
    dis_ref[...] = dis
    g0 = jnp.dot(x_ref[...], w1_ref[...],
                 preferred_element_type=jnp.float32) * dis
    gl_ref[...] = g0[:, :H]
    gr_ref[...] = g0[:, H:]


def _k4_body(al_ref, ar_ref, gl_ref, gr_ref, dis_ref, b1_ref, w2_ref,
             g1l_ref, g1r_ref):
    dis = dis_ref[...]
    hl = al_ref[...] + gl_ref[...]
    hr = ar_ref[...] + gr_ref[...]
    h1 = dis * jnp.concatenate([hl, hr], axis=1) + b1_ref[...]
    h1 = jnp.maximum(h1, 0.0)
    g1 = dis * jnp.dot(h1, w2_ref[...], preferred_element_type=jnp.float32)
    g1l_ref[...] = g1[:, :H]
    g1r_ref[...] = g1[:, H:]


def _k6_body(cl_ref, cr_ref, g1l_ref, g1r_ref, dis_ref, b2_ref,
             wfc1_ref, bfc1_ref, wfc2_ref, bfc2_ref, pred_ref, pc_ref):
    hl = cl_ref[...] + g1l_ref[...]
    hr = cr_ref[...] + g1r_ref[...]
    h2 = dis_ref[...] * jnp.concatenate([hl, hr], axis=1) + b2_ref[...]
    pred_ref[...] = (
        jnp.dot(h2, wfc1_ref[...], preferred_element_type=jnp.float32)
        + bfc1_ref[...]
    )
    pc_ref[...] = (
        jnp.dot(h2, wfc2_ref[...], preferred_element_type=jnp.float32)
        + bfc2_ref[...]
    )


def _row_block(minor):
    return pl.BlockSpec((B, minor), lambda i: (i, 0))


def _full_block(shape):
    return pl.BlockSpec(shape, lambda i: tuple(0 for _ in shape))


def kernel(x, edge_index, edge_weight, W1, b1, W2, b2, Wfc1, bfc1, Wfc2, bfc2):
    pad = EPT_PAD - EPT
    spread = jnp.broadcast_to(
        (jnp.arange(pad, dtype=jnp.int32) * 41) % N, (NS, pad))
    row_p = jnp.concatenate(
        [edge_index[0].reshape(NS, EPT), spread], axis=1)
    col_p = jnp.concatenate(
        [edge_index[1].reshape(NS, EPT), spread], axis=1)
    w_p = jnp.concatenate(
        [lax.bitcast_convert_type(edge_weight, jnp.int32).reshape(NS, EPT),
         jnp.zeros((NS, pad), jnp.int32)], axis=1)
    slab = jnp.stack([row_p, col_p, w_p], axis=1)
    slab = slab.reshape(NS, 3, MCH, 4, CA).transpose(0, 2, 3, 1, 4)
    col_w = edge_index[1].reshape(NW, CHD, C)
    w_w = edge_weight.reshape(NW, CHD, C)
    z_n = jnp.zeros((N,), jnp.float32)
    z_nh = jnp.zeros((N, H), jnp.float32)

    deg0, deg1 = _deg_kernel(col_w, w_w, z_n)

    grid = (N // B,)
    dis, g0l, g0r = pl.pallas_call(
        _k2_body,
        grid=grid,
        in_specs=[
            _row_block(1),
            _row_block(1),
            _row_block(F),
            _full_block((F, F)),
        ],
        out_specs=[_row_block(1), _row_block(H), _row_block(H)],
        out_shape=[
            jax.ShapeDtypeStruct((N, 1), jnp.float32),
            jax.ShapeDtypeStruct((N, H), jnp.float32),
            jax.ShapeDtypeStruct((N, H), jnp.float32),
        ],
    )(deg0[:, None], deg1[:, None], x, W1)

    a0l, a0r = _agg_kernel(slab, g0l, g0r, z_nh)

    g1l, g1r = pl.pallas_call(
        _k4_body,
        grid=grid,
        in_specs=[
            _row_block(H),
            _row_block(H),
            _row_block(H),
            _row_block(H),
            _row_block(1),
            _full_block((1, F)),
            _full_block((F, F)),
        ],
        out_specs=[_row_block(H), _row_block(H)],
        out_shape=[
            jax.ShapeDtypeStruct((N, H), jnp.float32),
            jax.ShapeDtypeStruct((N, H), jnp.float32),
        ],
    )(a0l, a0r, g0l, g0r, dis, b1[None, :], W2)

    a1l, a1r = _agg_kernel(slab, g1l, g1r, z_nh)

    pred, pred_cluster = pl.pallas_call(
        _k6_body,
        grid=grid,
        in_specs=[
            _row_block(H),
            _row_block(H),
            _row_block(H),
            _row_block(H),
            _row_block(1),
            _full_block((1, F)),
            _full_block((F, 64)),
            _full_block((1, 64)),
            _full_block((F, 16)),
            _full_block((1, 16)),
        ],
        out_specs=[_row_block(64), _row_block(16)],
        out_shape=[
            jax.ShapeDtypeStruct((N, 64), jnp.float32),
            jax.ShapeDtypeStruct((N, 16), jnp.float32),
        ],
    )(a1l, a1r, g1l, g1r, dis, b2[None, :], Wfc1, bfc1[None, :],
      Wfc2, bfc2[None, :])

    return (pred, pred_cluster)
